# baseline (speedup 1.0000x reference)
import functools

import jax
import jax.numpy as jnp
from jax import lax
from jax.experimental import pallas as pl
from jax.experimental.pallas import tpu as pltpu
from jax.experimental.pallas import tpu_sc as plsc

_L = 16
_W = 128


def _make_sc_kernel(B: int, C: int, NC: int, NS: int):
    NW = NC * NS
    per_w = B // NW
    n_blk = per_w // _W
    n_vec = per_w // _L
    mesh = plsc.VectorSubcoreMesh(core_axis_name="c", subcore_axis_name="s")

    @functools.partial(
        pl.kernel,
        mesh=mesh,
        out_type=jax.ShapeDtypeStruct((NW, _L), jnp.float32),
        compiler_params=pltpu.CompilerParams(needs_layout_passes=False),
        scratch_types=[
            pltpu.VMEM((per_w,), jnp.int32),
            pltpu.VMEM((per_w,), jnp.float32),
            pltpu.VMEM((per_w, _W), jnp.float32),
            pltpu.VMEM((_L,), jnp.float32),
            pltpu.SemaphoreType.DMA,
            pltpu.SemaphoreType.DMA,
        ],
    )
    def sc_kernel(pt_hbm, tgt_hbm, mar_hbm, out_hbm,
                  tgt_v, mar_v, buf_v, acc_v, sem, msem):
        wid = lax.axis_index("s") * NC + lax.axis_index("c")
        base = wid * per_w
        mar_cp = pltpu.async_copy(mar_hbm.at[pl.ds(base, per_w)], mar_v, msem)
        pltpu.sync_copy(tgt_hbm.at[pl.ds(base, per_w)], tgt_v)

        copies = []
        for sub in range(n_blk):
            copies.append(pltpu.async_copy(
                pt_hbm.at[tgt_v.at[pl.ds(sub * _W, _W)], pl.ds(base + sub * _W, _W)],
                buf_v.at[pl.ds(sub * _W, _W)],
                sem,
            ))
        mar_cp.wait()

        lane = lax.iota(jnp.int32, _L)
        steps_per_blk = _W // _L

        def extract_body(step, acc):
            rowc = step * _L + lane
            ln = lax.bitwise_and(rowc, _W - 1)
            g = plsc.load_gather(buf_v, [rowc, ln])
            m = mar_v[pl.ds(step * _L, _L)]
            w = jnp.exp(-0.5 * m * m)
            nz = (m > 0) | (m < 0)
            return acc + jnp.where(nz, w, 0.0) * g

        acc = jnp.zeros((_L,), jnp.float32)
        for sub in range(n_blk):
            copies[sub].wait()
            acc = lax.fori_loop(
                sub * steps_per_blk, (sub + 1) * steps_per_blk, extract_body, acc
            )
        acc_v[...] = acc
        pltpu.sync_copy(acc_v, out_hbm.at[wid])

    return sc_kernel


def kernel(preds, targets, margin):
    B, C = preds.shape
    info = plsc.get_sparse_core_info()
    NC, NS = info.num_cores, info.num_subcores
    sc_kernel = _make_sc_kernel(B, C, NC, NS)
    partials = sc_kernel(preds.T, targets.astype(jnp.int32), margin)
    return -jnp.sum(partials) / B
